# baseline (speedup 1.0000x reference)
import functools

import jax
import jax.numpy as jnp
from jax.experimental import pallas as pl
from jax.experimental.pallas import tpu as pltpu

_INTERPRET = False

NUM_GATES_K = 64
EPS_K = 1e-09
DIM_K = 4096
BN = 256


def _gate_kernel(nb, cap, x_ref, w_ref, t1_ref, g1_ref, i2_ref, p2_ref,
                 g2_ref, cnt_ref, prx_ref, carry1, carry2):
    i = pl.program_id(0)
    ni = jax.lax.rem(i, nb)
    E = NUM_GATES_K

    @pl.when(ni == 0)
    def _():
        carry1[...] = jnp.zeros_like(carry1)
        carry2[...] = jnp.zeros_like(carry2)
        cnt_ref[...] = jnp.zeros_like(cnt_ref)
        prx_ref[...] = jnp.zeros_like(prx_ref)

    xb = x_ref[0]
    w = w_ref[...]
    logits = jnp.dot(xb, w, preferred_element_type=jnp.float32)
    probs = jax.nn.softmax(logits, axis=-1)

    e_iota = jax.lax.broadcasted_iota(jnp.int32, (BN, E), 1)
    g1 = jnp.max(probs, axis=-1, keepdims=True)
    i1 = jnp.min(jnp.where(probs == g1, e_iota, E), axis=-1, keepdims=True)
    m1 = (e_iota == i1).astype(jnp.float32)
    pnt = probs * (1.0 - m1)
    g2 = jnp.max(pnt, axis=-1, keepdims=True)
    i2 = jnp.min(jnp.where(pnt == g2, e_iota, E), axis=-1, keepdims=True)
    m2 = (e_iota == i2).astype(jnp.float32)

    denom = g1 + g2 + EPS_K
    g1n = g1 / denom
    g2n = g2 / denom

    r_io = jax.lax.broadcasted_iota(jnp.int32, (BN, BN), 0)
    c_io = jax.lax.broadcasted_iota(jnp.int32, (BN, BN), 1)
    tri = (c_io < r_io).astype(jnp.float32)

    ex1 = jnp.dot(tri, m1, preferred_element_type=jnp.float32) + carry1[...]
    pos1 = jnp.sum(ex1 * m1, axis=-1, keepdims=True)
    kept1 = pos1 < float(cap)
    g1v = jnp.where(kept1, g1n, 0.0)
    t1 = jnp.where(kept1, i1 * cap + pos1.astype(jnp.int32), -1)

    ex2 = jnp.dot(tri, m2, preferred_element_type=jnp.float32) + carry2[...]
    p2 = jnp.sum(ex2 * m2, axis=-1, keepdims=True)

    carry1[...] = carry1[...] + jnp.sum(m1, axis=0, keepdims=True)
    carry2[...] = carry2[...] + jnp.sum(m2, axis=0, keepdims=True)
    cnt_ref[...] = cnt_ref[...] + jnp.sum(m1, axis=0)[None, None, :]
    prx_ref[...] = prx_ref[...] + jnp.sum(probs, axis=0)[None, None, :]

    t1_ref[...] = t1.reshape(1, 1, BN)
    g1_ref[...] = g1v.reshape(1, 1, BN)
    i2_ref[...] = i2.reshape(1, 1, BN)
    p2_ref[...] = p2.astype(jnp.int32).reshape(1, 1, BN)
    g2_ref[...] = g2n.reshape(1, 1, BN)


def _emit_kernel(nb, cap, n_tok, nbatch, t1_ref, g1_ref, i2_ref, p2_ref,
                 g2_ref, cntb_ref, cnt_ref, prx_ref, comb_ref, disp_ref,
                 loss_ref):
    i = pl.program_id(0)
    E = NUM_GATES_K
    capf = float(cap)

    t1 = t1_ref[0, 0, :].reshape(BN, 1)
    g1v = g1_ref[0, 0, :].reshape(BN, 1)
    i2 = i2_ref[0, 0, :].reshape(BN, 1)
    p2p = p2_ref[0, 0, :].reshape(BN, 1).astype(jnp.float32)
    g2n = g2_ref[0, 0, :].reshape(BN, 1)

    cnt_b = jnp.minimum(cntb_ref[0, 0, :], capf)
    e_iota = jax.lax.broadcasted_iota(jnp.int32, (BN, E), 1)
    m2 = e_iota == i2
    m1cnt = jnp.sum(jnp.where(m2, cnt_b[None, :], 0.0), axis=-1,
                    keepdims=True)
    pos2 = p2p + m1cnt
    kept2 = pos2 < capf
    g2v = jnp.where(kept2, g2n, 0.0)
    t2 = jnp.where(kept2, i2 * cap + pos2.astype(jnp.int32), -1)

    col = jax.lax.broadcasted_iota(jnp.int32, (BN, E * cap), 1)
    m1c = col == t1
    m2c = col == t2
    comb = jnp.where(m1c, g1v, 0.0) + jnp.where(m2c, g2v, 0.0)
    disp = (jnp.where(m1c, jnp.where(g1v > 0.0, 1.0, 0.0), 0.0)
            + jnp.where(m2c, jnp.where(g2v > 0.0, 1.0, 0.0), 0.0))
    comb_ref[0] = comb
    disp_ref[0] = disp

    @pl.when(i == 0)
    def _():
        s = jnp.sum(cnt_ref[...] * prx_ref[...])
        scale = float(E * E) / (float(n_tok) * float(n_tok) * nbatch * E)
        loss_ref[...] = (s * scale).reshape(1, 1)


@jax.jit
def kernel(x, w_gating):
    b, n, d = x.shape
    E = NUM_GATES_K
    cap = max(min(n, int(n * 1.25 / E)), 4)
    nb = n // BN
    grid = b * nb

    tok_spec = pl.BlockSpec((1, 1, BN), lambda i: (i, 0, 0))
    be_spec = pl.BlockSpec((1, 1, E), lambda i, nb=nb: (i // nb, 0, 0))

    outs_a = pl.pallas_call(
        functools.partial(_gate_kernel, nb, cap),
        grid=(grid,),
        in_specs=[
            pl.BlockSpec((1, BN, d), lambda i, nb=nb: (i // nb, i % nb, 0)),
            pl.BlockSpec((d, E), lambda i: (0, 0)),
        ],
        out_specs=[tok_spec, tok_spec, tok_spec, tok_spec, tok_spec,
                   be_spec, be_spec],
        out_shape=[
            jax.ShapeDtypeStruct((grid, 1, BN), jnp.int32),
            jax.ShapeDtypeStruct((grid, 1, BN), jnp.float32),
            jax.ShapeDtypeStruct((grid, 1, BN), jnp.int32),
            jax.ShapeDtypeStruct((grid, 1, BN), jnp.int32),
            jax.ShapeDtypeStruct((grid, 1, BN), jnp.float32),
            jax.ShapeDtypeStruct((b, 1, E), jnp.float32),
            jax.ShapeDtypeStruct((b, 1, E), jnp.float32),
        ],
        scratch_shapes=[pltpu.VMEM((1, E), jnp.float32),
                        pltpu.VMEM((1, E), jnp.float32)],
        interpret=_INTERPRET,
    )(x, w_gating)
    t1, g1v, i2, p2, g2n, cnt1, prx = outs_a

    full_be = pl.BlockSpec((b, 1, E), lambda i: (0, 0, 0))
    comb, disp, loss = pl.pallas_call(
        functools.partial(_emit_kernel, nb, cap, n, b),
        grid=(grid,),
        in_specs=[tok_spec, tok_spec, tok_spec, tok_spec, tok_spec,
                  be_spec, full_be, full_be],
        out_specs=[
            pl.BlockSpec((1, BN, E * cap), lambda i, nb=nb: (i // nb, i % nb, 0)),
            pl.BlockSpec((1, BN, E * cap), lambda i, nb=nb: (i // nb, i % nb, 0)),
            pl.BlockSpec((1, 1), lambda i: (0, 0)),
        ],
        out_shape=[
            jax.ShapeDtypeStruct((b, n, E * cap), jnp.float32),
            jax.ShapeDtypeStruct((b, n, E * cap), jnp.float32),
            jax.ShapeDtypeStruct((1, 1), jnp.float32),
        ],
        interpret=_INTERPRET,
    )(t1, g1v, i2, p2, g2n, cnt1, cnt1, prx)

    dispatch = disp.reshape(b, n, E, cap)
    combine = comb.reshape(b, n, E, cap)
    return dispatch, combine, loss.reshape(())


# speedup vs baseline: 1.0251x; 1.0251x over previous
import functools

import jax
import jax.numpy as jnp
from jax.experimental import pallas as pl
from jax.experimental.pallas import tpu as pltpu

_INTERPRET = False

NUM_GATES_K = 64
EPS_K = 1e-09
BN = 256


def _fused_kernel(nb, grid0, cap, n_tok, nbatch, x_ref, w_ref, comb_ref,
                  disp_ref, loss_ref, t1_s, g1_s, i2_s, p2_s, g2_s,
                  cnt_s, prx_s, carry1, carry2):
    i = pl.program_id(0)
    E = NUM_GATES_K
    capf = float(cap)

    @pl.when(i == 0)
    def _():
        cnt_s[...] = jnp.zeros_like(cnt_s)
        prx_s[...] = jnp.zeros_like(prx_s)

    @pl.when(jax.lax.rem(i, nb) == 0)
    def _():
        carry1[...] = jnp.zeros_like(carry1)
        carry2[...] = jnp.zeros_like(carry2)

    @pl.when(i < grid0)
    def _gate():
        bi = i // nb
        xb = x_ref[0]
        w = w_ref[...]
        logits = jnp.dot(xb, w, preferred_element_type=jnp.float32)
        probs = jax.nn.softmax(logits, axis=-1)

        e_iota = jax.lax.broadcasted_iota(jnp.int32, (BN, E), 1)
        g1 = jnp.max(probs, axis=-1, keepdims=True)
        i1 = jnp.min(jnp.where(probs == g1, e_iota, E), axis=-1, keepdims=True)
        m1 = (e_iota == i1).astype(jnp.float32)
        pnt = probs * (1.0 - m1)
        g2 = jnp.max(pnt, axis=-1, keepdims=True)
        i2 = jnp.min(jnp.where(pnt == g2, e_iota, E), axis=-1, keepdims=True)
        m2 = (e_iota == i2).astype(jnp.float32)

        denom = g1 + g2 + EPS_K
        g1n = g1 / denom
        g2n = g2 / denom

        r_io = jax.lax.broadcasted_iota(jnp.int32, (BN, BN), 0)
        c_io = jax.lax.broadcasted_iota(jnp.int32, (BN, BN), 1)
        tri = (c_io < r_io).astype(jnp.float32)

        ex1 = jnp.dot(tri, m1, preferred_element_type=jnp.float32) + carry1[...]
        pos1 = jnp.sum(ex1 * m1, axis=-1, keepdims=True)
        kept1 = pos1 < capf
        g1v = jnp.where(kept1, g1n, 0.0)
        t1 = jnp.where(kept1, i1 * cap + pos1.astype(jnp.int32), -1)

        ex2 = jnp.dot(tri, m2, preferred_element_type=jnp.float32) + carry2[...]
        p2 = jnp.sum(ex2 * m2, axis=-1, keepdims=True)

        carry1[...] = carry1[...] + jnp.sum(m1, axis=0, keepdims=True)
        carry2[...] = carry2[...] + jnp.sum(m2, axis=0, keepdims=True)
        cnt_s[bi] = cnt_s[bi] + jnp.sum(m1, axis=0, keepdims=True)
        prx_s[bi] = prx_s[bi] + jnp.sum(probs, axis=0, keepdims=True)

        t1_s[i] = t1.reshape(1, BN)
        g1_s[i] = g1v.reshape(1, BN)
        i2_s[i] = i2.reshape(1, BN)
        p2_s[i] = p2.reshape(1, BN)
        g2_s[i] = g2n.reshape(1, BN)

    @pl.when(i >= nb)
    def _emit():
        jb = i - nb
        t1 = t1_s[jb].reshape(BN, 1)
        g1v = g1_s[jb].reshape(BN, 1)
        i2 = i2_s[jb].reshape(BN, 1)
        p2p = p2_s[jb].reshape(BN, 1)
        g2n = g2_s[jb].reshape(BN, 1)

        cnt_b = jnp.minimum(cnt_s[jb // nb], capf)
        e_iota = jax.lax.broadcasted_iota(jnp.int32, (BN, E), 1)
        m2 = e_iota == i2
        m1cnt = jnp.sum(jnp.where(m2, cnt_b, 0.0), axis=-1, keepdims=True)
        pos2 = p2p + m1cnt
        kept2 = pos2 < capf
        g2v = jnp.where(kept2, g2n, 0.0)
        t2 = jnp.where(kept2, i2 * cap + pos2.astype(jnp.int32), -1)

        d1 = jnp.where(g1v > 0.0, 1.0, 0.0)
        d2 = jnp.where(g2v > 0.0, 1.0, 0.0)
        col = jax.lax.broadcasted_iota(jnp.int32, (BN, E * cap), 1)
        m1c = col == t1
        m2c = col == t2
        comb_ref[0] = jnp.where(m1c, g1v, jnp.where(m2c, g2v, 0.0))
        disp_ref[0] = jnp.where(m1c, d1, jnp.where(m2c, d2, 0.0))

    @pl.when(i == grid0 + nb - 1)
    def _loss():
        s = jnp.sum(cnt_s[...] * prx_s[...])
        scale = float(E * E) / (float(n_tok) * float(n_tok) * nbatch * E)
        loss_ref[...] = (s * scale).reshape(1, 1)


@jax.jit
def kernel(x, w_gating):
    b, n, d = x.shape
    E = NUM_GATES_K
    cap = max(min(n, int(n * 1.25 / E)), 4)
    nb = n // BN
    grid0 = b * nb
    grid = grid0 + nb

    def x_map(i, nb=nb, grid0=grid0):
        j = jnp.minimum(i, grid0 - 1)
        return (j // nb, j % nb, 0)

    def out_map(i, nb=nb):
        j = jnp.maximum(i - nb, 0)
        return (j // nb, j % nb, 0)

    comb, disp, loss = pl.pallas_call(
        functools.partial(_fused_kernel, nb, grid0, cap, n, b),
        grid=(grid,),
        in_specs=[
            pl.BlockSpec((1, BN, d), x_map),
            pl.BlockSpec((d, E), lambda i: (0, 0)),
        ],
        out_specs=[
            pl.BlockSpec((1, BN, E * cap), out_map),
            pl.BlockSpec((1, BN, E * cap), out_map),
            pl.BlockSpec((1, 1), lambda i: (0, 0)),
        ],
        out_shape=[
            jax.ShapeDtypeStruct((b, n, E * cap), jnp.float32),
            jax.ShapeDtypeStruct((b, n, E * cap), jnp.float32),
            jax.ShapeDtypeStruct((1, 1), jnp.float32),
        ],
        scratch_shapes=[
            pltpu.VMEM((grid0, 1, BN), jnp.int32),
            pltpu.VMEM((grid0, 1, BN), jnp.float32),
            pltpu.VMEM((grid0, 1, BN), jnp.int32),
            pltpu.VMEM((grid0, 1, BN), jnp.float32),
            pltpu.VMEM((grid0, 1, BN), jnp.float32),
            pltpu.VMEM((b, 1, E), jnp.float32),
            pltpu.VMEM((b, 1, E), jnp.float32),
            pltpu.VMEM((1, E), jnp.float32),
            pltpu.VMEM((1, E), jnp.float32),
        ],
        interpret=_INTERPRET,
    )(x, w_gating)

    dispatch = disp.reshape(b, n, E, cap)
    combine = comb.reshape(b, n, E, cap)
    return dispatch, combine, loss.reshape(())
